# baseline (speedup 1.0000x reference)
import functools

import jax
import jax.numpy as jnp
from jax import lax
from jax.experimental import pallas as pl
from jax.experimental.pallas import tpu as pltpu
from jax.experimental.pallas import tpu_sc as plsc

N = 50000
E = 800000
BATCH = 4
F_IN = 16
ROW = BATCH * F_IN
K_HOPS = 3
ORDER = 7
OUT_CH = 126
GRP_OUT = 18
EMB = 32
MLP_SIZE = 64
HORIZON = 12

N_PAD = 50176
NH = 25088
E_PAD = 802816
CH = 128

HALF = ROW // 2
OBR = 112

DEG_SPAN = N_PAD // 16


def _mesh():
    return plsc.VectorSubcoreMesh(core_axis_name="c", subcore_axis_name="s")


def _sc_params():
    return pltpu.CompilerParams(use_tc_tiling_on_sc=False)


def _deg_call(idx_f, idx_b, w):
    def body(if_hbm, ib_hbm, w_hbm, deg_f_out, deg_b_out, iv, wv, zb, db,
             acc_sh):
        c = lax.axis_index("c")
        s = lax.axis_index("s")

        def zero_body(r, _):
            zb[pl.ds(r * 16, 16)] = jnp.zeros((16,), jnp.float32)
            return 0

        lax.fori_loop(0, DEG_SPAN // 16, zero_body, 0)
        pltpu.sync_copy(zb, acc_sh.at[pl.ds(s * DEG_SPAN, DEG_SPAN)])
        plsc.subcore_barrier()

        def edge_body(it, _):
            e0 = s * (E_PAD // 16) + it * CH

            @pl.when(c == 0)
            def _():
                pltpu.sync_copy(if_hbm.at[pl.ds(e0, CH)], iv)

            @pl.when(c == 1)
            def _():
                pltpu.sync_copy(ib_hbm.at[pl.ds(e0, CH)], iv)

            pltpu.sync_copy(w_hbm.at[pl.ds(e0, CH)], wv)
            pltpu.sync_copy(wv, acc_sh.at[iv], add=True)
            return 0

        lax.fori_loop(0, E_PAD // 16 // CH, edge_body, 0)
        plsc.subcore_barrier()

        off = s * DEG_SPAN
        pltpu.sync_copy(acc_sh.at[pl.ds(off, DEG_SPAN)], db)

        @pl.when(c == 0)
        def _():
            pltpu.sync_copy(db, deg_f_out.at[pl.ds(off, DEG_SPAN)])

        @pl.when(c == 1)
        def _():
            pltpu.sync_copy(db, deg_b_out.at[pl.ds(off, DEG_SPAN)])

    call = functools.partial(
        pl.kernel,
        out_type=[jax.ShapeDtypeStruct((N_PAD,), jnp.float32),
                  jax.ShapeDtypeStruct((N_PAD,), jnp.float32)],
        mesh=_mesh(),
        compiler_params=_sc_params(),
        scratch_types=[
            pltpu.VMEM((CH,), jnp.int32),
            pltpu.VMEM((CH,), jnp.float32),
            pltpu.VMEM((DEG_SPAN,), jnp.float32),
            pltpu.VMEM((DEG_SPAN,), jnp.float32),
            pltpu.VMEM_SHARED((N_PAD,), jnp.float32),
        ],
    )(body)
    return call(idx_f, idx_b, w)


def _wn_call(idx, w, deg):
    epw = E_PAD // 32

    def body(idx_hbm, w_hbm, deg_hbm, wn_out, iv, wv, dv, wnv):
        c = lax.axis_index("c")
        s = lax.axis_index("s")
        wid = s * 2 + c

        def edge_body(it, _):
            e0 = wid * epw + it * CH
            pltpu.sync_copy(idx_hbm.at[pl.ds(e0, CH)], iv)
            pltpu.sync_copy(w_hbm.at[pl.ds(e0, CH)], wv)
            pltpu.sync_copy(deg_hbm.at[iv], dv)
            for i in range(CH // 16):
                sl = pl.ds(i * 16, 16)
                wnv[sl] = wv[sl] / jnp.maximum(dv[sl], 1e-8)
            pltpu.sync_copy(wnv, wn_out.at[pl.ds(e0, CH)])
            return 0

        lax.fori_loop(0, epw // CH, edge_body, 0)

    call = functools.partial(
        pl.kernel,
        out_type=jax.ShapeDtypeStruct((E_PAD,), jnp.float32),
        mesh=_mesh(),
        compiler_params=_sc_params(),
        scratch_types=[
            pltpu.VMEM((CH,), jnp.int32),
            pltpu.VMEM((CH,), jnp.float32),
            pltpu.VMEM((CH,), jnp.float32),
            pltpu.VMEM((CH,), jnp.float32),
        ],
    )(body)
    return call(idx, w, deg)


CPT = E_PAD // 16 // CH
NB = CPT // 4
NROW2 = E_PAD // CH


def _prop_call(h, gidx2, sidx2, wn2):
    def body(h_hbm, gi_hbm, si_hbm, wn_hbm, out_hbm,
             gia, gib, sia, sib, wna, wnb, lia, lib, rowsa, rowsb,
             ob, acc_sh, isema, isemb, gsema, gsemb, ssema, ssemb):
        c = lax.axis_index("c")
        s = lax.axis_index("s")
        hoff = c * N_PAD

        def zero_body(r, _):
            for j in range(HALF // 16):
                ob[r, pl.ds(j * 16, 16)] = jnp.zeros((16,), jnp.float32)
            return 0

        lax.fori_loop(0, OBR, zero_body, 0)
        for k in range(N_PAD // 16 // OBR):
            pltpu.sync_copy(
                ob, acc_sh.at[pl.ds(s * (N_PAD // 16) + k * OBR, OBR)])
        plsc.subcore_barrier()

        def fire_idx(b, gi, si, wn, sem):
            rb = s * CPT + jnp.minimum(2 * b, CPT - 2)
            pltpu.async_copy(gi_hbm.at[pl.ds(rb, 2)], gi, sem)
            pltpu.async_copy(si_hbm.at[pl.ds(rb, 2)], si, sem)
            pltpu.async_copy(wn_hbm.at[pl.ds(rb, 2)], wn, sem)

        def drain_idx(gi, si, wn, sem):
            pltpu.make_async_copy(gi_hbm.at[pl.ds(0, 2)], gi, sem).wait()
            pltpu.make_async_copy(si_hbm.at[pl.ds(0, 2)], si, sem).wait()
            pltpu.make_async_copy(wn_hbm.at[pl.ds(0, 2)], wn, sem).wait()

        def offset_idx(gi):
            for j in range(2):
                for i in range(8):
                    sl = pl.ds(i * 16, 16)
                    gi[j, sl] = gi[j, sl] + hoff

        def fire_gather(gi, rows, sem):
            for j in range(2):
                pltpu.async_copy(h_hbm.at[gi.at[j]],
                                 rows.at[pl.ds(j * CH, CH)], sem)

        def drain_gather(gi, rows, sem):
            for j in range(2):
                pltpu.make_async_copy(h_hbm.at[gi.at[0]],
                                      rows.at[pl.ds(0, CH)], sem).wait()

        def fire_scatter(si, rows, sem):
            for j in range(2):
                pltpu.async_copy(rows.at[pl.ds(j * CH, CH)],
                                 acc_sh.at[si.at[j]], sem, add=True)

        def drain_scatter(si, rows, sem):
            for j in range(2):
                pltpu.make_async_copy(rows.at[pl.ds(0, CH)],
                                      acc_sh.at[si.at[0]], sem).wait()

        def compute(si, li, wn, rows):
            for j in range(2):
                for i in range(8):
                    sl = pl.ds(i * 16, 16)
                    li[j, sl] = si[j, sl]
            for j in range(2):
                for g in range(8):
                    wvec = wn[j, pl.ds(g * 16, 16)]
                    for kk in range(16):
                        r = j * CH + g * 16 + kk
                        wsc = wvec[kk]
                        for m in range(HALF // 16):
                            sl2 = pl.ds(m * 16, 16)
                            rows[r, sl2] = rows[r, sl2] * wsc

        fire_idx(0, gia, sia, wna, isema)
        fire_idx(1, gib, sib, wnb, isemb)
        drain_idx(gia, sia, wna, isema)
        offset_idx(gia)
        fire_gather(gia, rowsa, gsema)
        for j in range(2):
            for i in range(8):
                lib[j, pl.ds(i * 16, 16)] = jnp.zeros((16,), jnp.int32)

        def zrows_body(r, _):
            for m in range(HALF // 16):
                rowsb[r, pl.ds(m * 16, 16)] = jnp.zeros((16,), jnp.float32)
            return 0

        lax.fori_loop(0, 2 * CH, zrows_body, 0)
        fire_scatter(lib, rowsb, ssemb)

        def pair_body(t, _):
            b0 = 2 * t
            drain_gather(gia, rowsa, gsema)
            compute(sia, lia, wna, rowsa)
            fire_scatter(lia, rowsa, ssema)
            fire_idx(b0 + 2, gia, sia, wna, isema)
            drain_idx(gib, sib, wnb, isemb)
            offset_idx(gib)
            drain_scatter(lib, rowsb, ssemb)
            fire_gather(gib, rowsb, gsemb)
            drain_gather(gib, rowsb, gsemb)
            compute(sib, lib, wnb, rowsb)
            fire_scatter(lib, rowsb, ssemb)
            fire_idx(b0 + 3, gib, sib, wnb, isemb)
            drain_idx(gia, sia, wna, isema)
            offset_idx(gia)
            drain_scatter(lia, rowsa, ssema)
            fire_gather(gia, rowsa, gsema)
            return 0

        lax.fori_loop(0, CPT // 4, pair_body, 0)

        drain_gather(gia, rowsa, gsema)
        drain_scatter(lib, rowsb, ssemb)
        drain_idx(gib, sib, wnb, isemb)
        plsc.subcore_barrier()

        for k in range(N_PAD // 16 // OBR):
            off = s * (N_PAD // 16) + k * OBR
            pltpu.sync_copy(acc_sh.at[pl.ds(off, OBR)], ob)
            pltpu.sync_copy(ob, out_hbm.at[pl.ds(hoff + off, OBR)])

    call = functools.partial(
        pl.kernel,
        out_type=jax.ShapeDtypeStruct((2 * N_PAD, HALF), jnp.float32),
        mesh=_mesh(),
        compiler_params=_sc_params(),
        scratch_types=[
            pltpu.VMEM((2, CH), jnp.int32),
            pltpu.VMEM((2, CH), jnp.int32),
            pltpu.VMEM((2, CH), jnp.int32),
            pltpu.VMEM((2, CH), jnp.int32),
            pltpu.VMEM((2, CH), jnp.float32),
            pltpu.VMEM((2, CH), jnp.float32),
            pltpu.VMEM((2, CH), jnp.int32),
            pltpu.VMEM((2, CH), jnp.int32),
            pltpu.VMEM((2 * CH, HALF), jnp.float32),
            pltpu.VMEM((2 * CH, HALF), jnp.float32),
            pltpu.VMEM((OBR, HALF), jnp.float32),
            pltpu.VMEM_SHARED((N_PAD, HALF), jnp.float32),
            pltpu.SemaphoreType.DMA,
            pltpu.SemaphoreType.DMA,
            pltpu.SemaphoreType.DMA,
            pltpu.SemaphoreType.DMA,
            pltpu.SemaphoreType.DMA,
            pltpu.SemaphoreType.DMA,
        ],
    )(body)
    return call(h, gidx2, sidx2, wn2)


def _prep_weights(conv_w, conv_b, lin_w, lin_b, mlp_w, mlp_b, ro_w, ro_b):
    wb = jnp.zeros((ORDER * ROW, BATCH * OUT_CH), jnp.float32)
    for g in range(ORDER):
        blk = conv_w[g].T
        for b in range(BATCH):
            wb = wb.at[g * ROW + b * F_IN: g * ROW + (b + 1) * F_IN,
                       b * OUT_CH + g * GRP_OUT: b * OUT_CH + (g + 1) * GRP_OUT
                       ].set(blk)
    cb = jnp.tile(conv_b, BATCH)[None, :]
    linb = jnp.tile(lin_w.T, (1, BATCH))
    lb = jnp.tile(lin_b, BATCH)[None, :]
    mlpb = jnp.zeros((BATCH * OUT_CH, BATCH * MLP_SIZE), jnp.float32)
    rob = jnp.zeros((BATCH * MLP_SIZE, BATCH * HORIZON), jnp.float32)
    for b in range(BATCH):
        mlpb = mlpb.at[b * OUT_CH:(b + 1) * OUT_CH,
                       b * MLP_SIZE:(b + 1) * MLP_SIZE].set(mlp_w.T)
        rob = rob.at[b * MLP_SIZE:(b + 1) * MLP_SIZE,
                     b * HORIZON:(b + 1) * HORIZON].set(ro_w.T)
    mb = jnp.tile(mlp_b, BATCH)[None, :]
    rb = jnp.tile(ro_b, BATCH)[None, :]
    return wb, cb, linb, lb, mlpb, mb, rob, rb


def _silu(v):
    return v * jax.nn.sigmoid(v)


def _dense_call(hs, embp, wb, cb, linb, lb, mlpb, mb, rob, rb):
    BLK = 1024
    grid = N_PAD // BLK

    def body(x0l, x0h, x1l, x1h, x2l, x2h, x3l, x3h, x4l, x4h, x5l, x5h,
             x6l, x6h, e, wbr, cbr, linr, lbr, mlpr, mbr, ror, rbr, o):
        z = jnp.concatenate(
            [x0l[...], x0h[...], x1l[...], x1h[...], x2l[...], x2h[...],
             x3l[...], x3h[...], x4l[...], x4h[...], x5l[...], x5h[...],
             x6l[...], x6h[...]], axis=1)
        enc = jnp.dot(z, wbr[...], preferred_element_type=jnp.float32)
        enc = _silu(enc + cbr[...])
        enc = enc + jnp.dot(e[...], linr[...],
                            preferred_element_type=jnp.float32) + lbr[...]
        hm = _silu(jnp.dot(enc, mlpr[...],
                           preferred_element_type=jnp.float32) + mbr[...])
        o[...] = jnp.dot(hm, ror[...],
                         preferred_element_type=jnp.float32) + rbr[...]

    spec_lo = pl.BlockSpec((BLK, HALF), lambda i: (i, 0))
    spec_hi = pl.BlockSpec((BLK, HALF), lambda i: (i + N_PAD // BLK, 0))

    def fullspec(a):
        return pl.BlockSpec(a.shape, lambda i: tuple(0 for _ in a.shape))

    xargs = []
    xspecs = []
    for hk in hs:
        xargs += [hk, hk]
        xspecs += [spec_lo, spec_hi]

    out = pl.pallas_call(
        body,
        grid=(grid,),
        in_specs=xspecs + [pl.BlockSpec((BLK, EMB), lambda i: (i, 0))] +
                 [fullspec(a) for a in (wb, cb, linb, lb, mlpb, mb, rob, rb)],
        out_specs=pl.BlockSpec((BLK, BATCH * HORIZON), lambda i: (i, 0)),
        out_shape=jax.ShapeDtypeStruct((N_PAD, BATCH * HORIZON), jnp.float32),
    )(*xargs, embp, wb, cb, linb, lb, mlpb, mb, rob, rb)
    return out


def kernel(x, edge_index, edge_weight, conv_w, conv_b, emb, lin_w, lin_b,
           mlp_w, mlp_b, ro_w, ro_b):
    xt = x[:, -1]
    lo = xt[0:2].transpose(1, 0, 2).reshape(N, HALF)
    hi = xt[2:4].transpose(1, 0, 2).reshape(N, HALF)
    h0 = jnp.concatenate([jnp.pad(lo, ((0, N_PAD - N), (0, 0))),
                          jnp.pad(hi, ((0, N_PAD - N), (0, 0)))], axis=0)
    src = jnp.pad(edge_index[0].astype(jnp.int32), (0, E_PAD - E))
    dst = jnp.pad(edge_index[1].astype(jnp.int32), (0, E_PAD - E))
    w = jnp.pad(edge_weight, (0, E_PAD - E))

    deg_f, deg_b = _deg_call(dst, src, w)
    wn_f = _wn_call(dst, w, deg_f)
    wn_b = _wn_call(src, w, deg_b)

    src2 = src.reshape(NROW2, CH)
    dst2 = dst.reshape(NROW2, CH)
    wn_f2 = wn_f.reshape(NROW2, CH)
    wn_b2 = wn_b.reshape(NROW2, CH)
    h1 = _prop_call(h0, src2, dst2, wn_f2)
    h2 = _prop_call(h1, src2, dst2, wn_f2)
    h3 = _prop_call(h2, src2, dst2, wn_f2)
    h4 = _prop_call(h0, dst2, src2, wn_b2)
    h5 = _prop_call(h4, dst2, src2, wn_b2)
    h6 = _prop_call(h5, dst2, src2, wn_b2)

    wts = _prep_weights(conv_w, conv_b, lin_w, lin_b, mlp_w, mlp_b, ro_w, ro_b)
    embp = jnp.pad(emb, ((0, N_PAD - N), (0, 0)))
    out48 = _dense_call((h0, h1, h2, h3, h4, h5, h6), embp, *wts)

    out = out48[:N].reshape(N, BATCH, HORIZON).transpose(1, 2, 0)[..., None]
    return out


# speedup vs baseline: 1.3831x; 1.3831x over previous
import functools

import jax
import jax.numpy as jnp
from jax import lax
from jax.experimental import pallas as pl
from jax.experimental.pallas import tpu as pltpu
from jax.experimental.pallas import tpu_sc as plsc

N = 50000
E = 800000
BATCH = 4
F_IN = 16
ROW = BATCH * F_IN
K_HOPS = 3
ORDER = 7
OUT_CH = 126
GRP_OUT = 18
EMB = 32
MLP_SIZE = 64
HORIZON = 12

N_PAD = 50176
NH = 25088
E_PAD = 802816
CH = 128

HALF = ROW // 2
OBR = 112

DEG_SPAN = N_PAD // 16


def _mesh():
    return plsc.VectorSubcoreMesh(core_axis_name="c", subcore_axis_name="s")


def _sc_params():
    return pltpu.CompilerParams(use_tc_tiling_on_sc=False)


DCH = 7


def _deg_call(idx_f2, idx_b2, w2):
    steps = CPT // DCH

    def body(if_hbm, ib_hbm, w_hbm, deg_f_out, deg_b_out,
             iva, ivb, wva, wvb, zb, db, acc_sh,
             isema, isemb, ssema, ssemb):
        c = lax.axis_index("c")
        s = lax.axis_index("s")

        def zero_body(r, _):
            zb[pl.ds(r * 16, 16)] = jnp.zeros((16,), jnp.float32)
            return 0

        lax.fori_loop(0, DEG_SPAN // 16, zero_body, 0)
        pltpu.sync_copy(zb, acc_sh.at[pl.ds(s * DEG_SPAN, DEG_SPAN)])
        plsc.subcore_barrier()

        def fireI(b, iv, wv, sem):
            rb = s * CPT + jnp.minimum(DCH * b, CPT - DCH)

            @pl.when(c == 0)
            def _():
                pltpu.async_copy(if_hbm.at[pl.ds(rb, DCH)], iv, sem)

            @pl.when(c == 1)
            def _():
                pltpu.async_copy(ib_hbm.at[pl.ds(rb, DCH)], iv, sem)

            pltpu.async_copy(w_hbm.at[pl.ds(rb, DCH)], wv, sem)

        def drainI(iv, wv, sem):
            pltpu.make_async_copy(if_hbm.at[pl.ds(0, DCH)], iv, sem).wait()
            pltpu.make_async_copy(w_hbm.at[pl.ds(0, DCH)], wv, sem).wait()

        def fireS(iv, wv, sem):
            for j in range(DCH):
                pltpu.async_copy(wv.at[j], acc_sh.at[iv.at[j]], sem, add=True)

        def drainS(iv, wv, sem):
            for j in range(DCH):
                pltpu.make_async_copy(wv.at[0], acc_sh.at[iv.at[0]],
                                      sem).wait()

        fireI(0, iva, wva, isema)
        fireI(1, ivb, wvb, isemb)

        def pair_body(t, _):
            s0 = 2 * t
            drainI(iva, wva, isema)
            fireS(iva, wva, ssema)
            drainI(ivb, wvb, isemb)
            fireS(ivb, wvb, ssemb)
            drainS(iva, wva, ssema)
            fireI(s0 + 2, iva, wva, isema)
            drainS(ivb, wvb, ssemb)
            fireI(s0 + 3, ivb, wvb, isemb)
            return 0

        lax.fori_loop(0, steps // 2, pair_body, 0)
        drainI(iva, wva, isema)
        drainI(ivb, wvb, isemb)
        plsc.subcore_barrier()

        off = s * DEG_SPAN
        pltpu.sync_copy(acc_sh.at[pl.ds(off, DEG_SPAN)], db)

        @pl.when(c == 0)
        def _():
            pltpu.sync_copy(db, deg_f_out.at[pl.ds(off, DEG_SPAN)])

        @pl.when(c == 1)
        def _():
            pltpu.sync_copy(db, deg_b_out.at[pl.ds(off, DEG_SPAN)])

    call = functools.partial(
        pl.kernel,
        out_type=[jax.ShapeDtypeStruct((N_PAD,), jnp.float32),
                  jax.ShapeDtypeStruct((N_PAD,), jnp.float32)],
        mesh=_mesh(),
        compiler_params=_sc_params(),
        scratch_types=[
            pltpu.VMEM((DCH, CH), jnp.int32),
            pltpu.VMEM((DCH, CH), jnp.int32),
            pltpu.VMEM((DCH, CH), jnp.float32),
            pltpu.VMEM((DCH, CH), jnp.float32),
            pltpu.VMEM((DEG_SPAN,), jnp.float32),
            pltpu.VMEM((DEG_SPAN,), jnp.float32),
            pltpu.VMEM_SHARED((N_PAD,), jnp.float32),
            pltpu.SemaphoreType.DMA,
            pltpu.SemaphoreType.DMA,
            pltpu.SemaphoreType.DMA,
            pltpu.SemaphoreType.DMA,
        ],
    )(body)
    return call(idx_f2, idx_b2, w2)


def _wn_call(dst2, src2, w2, deg_f, deg_b):
    cpw = NROW2 // 32
    steps = cpw // DCH

    def body(dst_hbm, src_hbm, w_hbm, degf_hbm, degb_hbm,
             wnf_out, wnb_out,
             dva, dvb, sva, svb, wva, wvb, gfa, gfb, gba, gbb,
             ofa, ofb, oba, obb, isema, isemb, gsema, gsemb, osema, osemb):
        c = lax.axis_index("c")
        s = lax.axis_index("s")
        wid = s * 2 + c

        def rbase(b):
            return wid * cpw + jnp.minimum(DCH * b, cpw - DCH)

        def fireI(b, dv, sv, wv, sem):
            rb = rbase(b)
            pltpu.async_copy(dst_hbm.at[pl.ds(rb, DCH)], dv, sem)
            pltpu.async_copy(src_hbm.at[pl.ds(rb, DCH)], sv, sem)
            pltpu.async_copy(w_hbm.at[pl.ds(rb, DCH)], wv, sem)

        def drainI(dv, sv, wv, sem):
            pltpu.make_async_copy(dst_hbm.at[pl.ds(0, DCH)], dv, sem).wait()
            pltpu.make_async_copy(src_hbm.at[pl.ds(0, DCH)], sv, sem).wait()
            pltpu.make_async_copy(w_hbm.at[pl.ds(0, DCH)], wv, sem).wait()

        def fireG(dv, sv, gf, gb, sem):
            for j in range(DCH):
                pltpu.async_copy(degf_hbm.at[dv.at[j]], gf.at[j], sem)
                pltpu.async_copy(degb_hbm.at[sv.at[j]], gb.at[j], sem)

        def drainG(dv, sv, gf, gb, sem):
            for j in range(DCH):
                pltpu.make_async_copy(degf_hbm.at[dv.at[0]], gf.at[0],
                                      sem).wait()
                pltpu.make_async_copy(degb_hbm.at[sv.at[0]], gb.at[0],
                                      sem).wait()

        def compute(wv, gf, gb, of, ob):
            for j in range(DCH):
                for i in range(8):
                    sl = pl.ds(i * 16, 16)
                    of[j, sl] = wv[j, sl] / jnp.maximum(gf[j, sl], 1e-8)
                    ob[j, sl] = wv[j, sl] / jnp.maximum(gb[j, sl], 1e-8)

        def fireO(b, of, ob, sem):
            rb = rbase(b)
            pltpu.async_copy(of, wnf_out.at[pl.ds(rb, DCH)], sem)
            pltpu.async_copy(ob, wnb_out.at[pl.ds(rb, DCH)], sem)

        def drainO(of, ob, sem):
            pltpu.make_async_copy(of, wnf_out.at[pl.ds(0, DCH)], sem).wait()
            pltpu.make_async_copy(ob, wnb_out.at[pl.ds(0, DCH)], sem).wait()

        fireI(0, dva, sva, wva, isema)
        fireI(1, dvb, svb, wvb, isemb)

        def pair_body(t, _):
            s0 = 2 * t
            drainI(dva, sva, wva, isema)
            fireG(dva, sva, gfa, gba, gsema)
            drainI(dvb, svb, wvb, isemb)
            fireG(dvb, svb, gfb, gbb, gsemb)
            drainG(dva, sva, gfa, gba, gsema)

            @pl.when(t > 0)
            def _():
                drainO(ofa, oba, osema)

            compute(wva, gfa, gba, ofa, oba)
            fireO(s0, ofa, oba, osema)
            fireI(s0 + 2, dva, sva, wva, isema)
            drainG(dvb, svb, gfb, gbb, gsemb)

            @pl.when(t > 0)
            def _():
                drainO(ofb, obb, osemb)

            compute(wvb, gfb, gbb, ofb, obb)
            fireO(s0 + 1, ofb, obb, osemb)
            fireI(s0 + 3, dvb, svb, wvb, isemb)
            return 0

        lax.fori_loop(0, steps // 2, pair_body, 0)
        drainI(dva, sva, wva, isema)
        drainI(dvb, svb, wvb, isemb)
        drainO(ofa, oba, osema)
        drainO(ofb, obb, osemb)

    call = functools.partial(
        pl.kernel,
        out_type=[jax.ShapeDtypeStruct((NROW2, CH), jnp.float32),
                  jax.ShapeDtypeStruct((NROW2, CH), jnp.float32)],
        mesh=_mesh(),
        compiler_params=_sc_params(),
        scratch_types=(
            [pltpu.VMEM((DCH, CH), jnp.int32)] * 4 +
            [pltpu.VMEM((DCH, CH), jnp.float32)] * 10 +
            [pltpu.SemaphoreType.DMA] * 6
        ),
    )(body)
    return call(dst2, src2, w2, deg_f, deg_b)


CPT = E_PAD // 16 // CH
NB = CPT // 4
NROW2 = E_PAD // CH


def _prop_call(h, gidx2, sidx2, wn2):
    def body(h_hbm, gi_hbm, si_hbm, wn_hbm, out_hbm,
             gia, gib, sia, sib, wna, wnb, lia, lib, rowsa, rowsb,
             ob, acc_sh, isema, isemb, gsema, gsemb, ssema, ssemb):
        c = lax.axis_index("c")
        s = lax.axis_index("s")
        hoff = c * N_PAD

        def zero_body(r, _):
            for j in range(HALF // 16):
                ob[r, pl.ds(j * 16, 16)] = jnp.zeros((16,), jnp.float32)
            return 0

        lax.fori_loop(0, OBR, zero_body, 0)
        for k in range(N_PAD // 16 // OBR):
            pltpu.sync_copy(
                ob, acc_sh.at[pl.ds(s * (N_PAD // 16) + k * OBR, OBR)])
        plsc.subcore_barrier()

        def fire_idx(b, gi, si, wn, sem):
            rb = s * CPT + jnp.minimum(2 * b, CPT - 2)
            pltpu.async_copy(gi_hbm.at[pl.ds(rb, 2)], gi, sem)
            pltpu.async_copy(si_hbm.at[pl.ds(rb, 2)], si, sem)
            pltpu.async_copy(wn_hbm.at[pl.ds(rb, 2)], wn, sem)

        def drain_idx(gi, si, wn, sem):
            pltpu.make_async_copy(gi_hbm.at[pl.ds(0, 2)], gi, sem).wait()
            pltpu.make_async_copy(si_hbm.at[pl.ds(0, 2)], si, sem).wait()
            pltpu.make_async_copy(wn_hbm.at[pl.ds(0, 2)], wn, sem).wait()

        def offset_idx(gi):
            for j in range(2):
                for i in range(8):
                    sl = pl.ds(i * 16, 16)
                    gi[j, sl] = gi[j, sl] + hoff

        def fire_gather(gi, rows, sem):
            for j in range(2):
                pltpu.async_copy(h_hbm.at[gi.at[j]],
                                 rows.at[pl.ds(j * CH, CH)], sem)

        def drain_gather(gi, rows, sem):
            for j in range(2):
                pltpu.make_async_copy(h_hbm.at[gi.at[0]],
                                      rows.at[pl.ds(0, CH)], sem).wait()

        def fire_scatter(si, rows, sem):
            for j in range(2):
                pltpu.async_copy(rows.at[pl.ds(j * CH, CH)],
                                 acc_sh.at[si.at[j]], sem, add=True)

        def drain_scatter(si, rows, sem):
            for j in range(2):
                pltpu.make_async_copy(rows.at[pl.ds(0, CH)],
                                      acc_sh.at[si.at[0]], sem).wait()

        def compute(si, li, wn, rows):
            for j in range(2):
                for i in range(8):
                    sl = pl.ds(i * 16, 16)
                    li[j, sl] = si[j, sl]
            for j in range(2):
                for g in range(8):
                    wvec = wn[j, pl.ds(g * 16, 16)]
                    for kk in range(16):
                        r = j * CH + g * 16 + kk
                        wsc = wvec[kk]
                        for m in range(HALF // 16):
                            sl2 = pl.ds(m * 16, 16)
                            rows[r, sl2] = rows[r, sl2] * wsc

        fire_idx(0, gia, sia, wna, isema)
        fire_idx(1, gib, sib, wnb, isemb)
        drain_idx(gia, sia, wna, isema)
        offset_idx(gia)
        fire_gather(gia, rowsa, gsema)
        for j in range(2):
            for i in range(8):
                lib[j, pl.ds(i * 16, 16)] = jnp.zeros((16,), jnp.int32)

        def zrows_body(r, _):
            for m in range(HALF // 16):
                rowsb[r, pl.ds(m * 16, 16)] = jnp.zeros((16,), jnp.float32)
            return 0

        lax.fori_loop(0, 2 * CH, zrows_body, 0)
        fire_scatter(lib, rowsb, ssemb)

        def pair_body(t, _):
            b0 = 2 * t
            drain_gather(gia, rowsa, gsema)
            compute(sia, lia, wna, rowsa)
            fire_scatter(lia, rowsa, ssema)
            fire_idx(b0 + 2, gia, sia, wna, isema)
            drain_idx(gib, sib, wnb, isemb)
            offset_idx(gib)
            drain_scatter(lib, rowsb, ssemb)
            fire_gather(gib, rowsb, gsemb)
            drain_gather(gib, rowsb, gsemb)
            compute(sib, lib, wnb, rowsb)
            fire_scatter(lib, rowsb, ssemb)
            fire_idx(b0 + 3, gib, sib, wnb, isemb)
            drain_idx(gia, sia, wna, isema)
            offset_idx(gia)
            drain_scatter(lia, rowsa, ssema)
            fire_gather(gia, rowsa, gsema)
            return 0

        lax.fori_loop(0, CPT // 4, pair_body, 0)

        drain_gather(gia, rowsa, gsema)
        drain_scatter(lib, rowsb, ssemb)
        drain_idx(gib, sib, wnb, isemb)
        plsc.subcore_barrier()

        for k in range(N_PAD // 16 // OBR):
            off = s * (N_PAD // 16) + k * OBR
            pltpu.sync_copy(acc_sh.at[pl.ds(off, OBR)], ob)
            pltpu.sync_copy(ob, out_hbm.at[pl.ds(hoff + off, OBR)])

    call = functools.partial(
        pl.kernel,
        out_type=jax.ShapeDtypeStruct((2 * N_PAD, HALF), jnp.float32),
        mesh=_mesh(),
        compiler_params=_sc_params(),
        scratch_types=[
            pltpu.VMEM((2, CH), jnp.int32),
            pltpu.VMEM((2, CH), jnp.int32),
            pltpu.VMEM((2, CH), jnp.int32),
            pltpu.VMEM((2, CH), jnp.int32),
            pltpu.VMEM((2, CH), jnp.float32),
            pltpu.VMEM((2, CH), jnp.float32),
            pltpu.VMEM((2, CH), jnp.int32),
            pltpu.VMEM((2, CH), jnp.int32),
            pltpu.VMEM((2 * CH, HALF), jnp.float32),
            pltpu.VMEM((2 * CH, HALF), jnp.float32),
            pltpu.VMEM((OBR, HALF), jnp.float32),
            pltpu.VMEM_SHARED((N_PAD, HALF), jnp.float32),
            pltpu.SemaphoreType.DMA,
            pltpu.SemaphoreType.DMA,
            pltpu.SemaphoreType.DMA,
            pltpu.SemaphoreType.DMA,
            pltpu.SemaphoreType.DMA,
            pltpu.SemaphoreType.DMA,
        ],
    )(body)
    return call(h, gidx2, sidx2, wn2)


def _prep_weights(conv_w, conv_b, lin_w, lin_b, mlp_w, mlp_b, ro_w, ro_b):
    wb = jnp.zeros((ORDER * ROW, BATCH * OUT_CH), jnp.float32)
    for g in range(ORDER):
        blk = conv_w[g].T
        for b in range(BATCH):
            wb = wb.at[g * ROW + b * F_IN: g * ROW + (b + 1) * F_IN,
                       b * OUT_CH + g * GRP_OUT: b * OUT_CH + (g + 1) * GRP_OUT
                       ].set(blk)
    cb = jnp.tile(conv_b, BATCH)[None, :]
    linb = jnp.tile(lin_w.T, (1, BATCH))
    lb = jnp.tile(lin_b, BATCH)[None, :]
    mlpb = jnp.zeros((BATCH * OUT_CH, BATCH * MLP_SIZE), jnp.float32)
    rob = jnp.zeros((BATCH * MLP_SIZE, BATCH * HORIZON), jnp.float32)
    for b in range(BATCH):
        mlpb = mlpb.at[b * OUT_CH:(b + 1) * OUT_CH,
                       b * MLP_SIZE:(b + 1) * MLP_SIZE].set(mlp_w.T)
        rob = rob.at[b * MLP_SIZE:(b + 1) * MLP_SIZE,
                     b * HORIZON:(b + 1) * HORIZON].set(ro_w.T)
    mb = jnp.tile(mlp_b, BATCH)[None, :]
    rb = jnp.tile(ro_b, BATCH)[None, :]
    return wb, cb, linb, lb, mlpb, mb, rob, rb


def _silu(v):
    return v * jax.nn.sigmoid(v)


def _dense_call(hs, embp, wb, cb, linb, lb, mlpb, mb, rob, rb):
    BLK = 1024
    grid = N_PAD // BLK

    def body(x0l, x0h, x1l, x1h, x2l, x2h, x3l, x3h, x4l, x4h, x5l, x5h,
             x6l, x6h, e, wbr, cbr, linr, lbr, mlpr, mbr, ror, rbr, o):
        z = jnp.concatenate(
            [x0l[...], x0h[...], x1l[...], x1h[...], x2l[...], x2h[...],
             x3l[...], x3h[...], x4l[...], x4h[...], x5l[...], x5h[...],
             x6l[...], x6h[...]], axis=1)
        enc = jnp.dot(z, wbr[...], preferred_element_type=jnp.float32)
        enc = _silu(enc + cbr[...])
        enc = enc + jnp.dot(e[...], linr[...],
                            preferred_element_type=jnp.float32) + lbr[...]
        hm = _silu(jnp.dot(enc, mlpr[...],
                           preferred_element_type=jnp.float32) + mbr[...])
        o[...] = jnp.dot(hm, ror[...],
                         preferred_element_type=jnp.float32) + rbr[...]

    spec_lo = pl.BlockSpec((BLK, HALF), lambda i: (i, 0))
    spec_hi = pl.BlockSpec((BLK, HALF), lambda i: (i + N_PAD // BLK, 0))

    def fullspec(a):
        return pl.BlockSpec(a.shape, lambda i: tuple(0 for _ in a.shape))

    xargs = []
    xspecs = []
    for hk in hs:
        xargs += [hk, hk]
        xspecs += [spec_lo, spec_hi]

    out = pl.pallas_call(
        body,
        grid=(grid,),
        in_specs=xspecs + [pl.BlockSpec((BLK, EMB), lambda i: (i, 0))] +
                 [fullspec(a) for a in (wb, cb, linb, lb, mlpb, mb, rob, rb)],
        out_specs=pl.BlockSpec((BLK, BATCH * HORIZON), lambda i: (i, 0)),
        out_shape=jax.ShapeDtypeStruct((N_PAD, BATCH * HORIZON), jnp.float32),
    )(*xargs, embp, wb, cb, linb, lb, mlpb, mb, rob, rb)
    return out


def kernel(x, edge_index, edge_weight, conv_w, conv_b, emb, lin_w, lin_b,
           mlp_w, mlp_b, ro_w, ro_b):
    xt = x[:, -1]
    lo = xt[0:2].transpose(1, 0, 2).reshape(N, HALF)
    hi = xt[2:4].transpose(1, 0, 2).reshape(N, HALF)
    h0 = jnp.concatenate([jnp.pad(lo, ((0, N_PAD - N), (0, 0))),
                          jnp.pad(hi, ((0, N_PAD - N), (0, 0)))], axis=0)
    src = jnp.pad(edge_index[0].astype(jnp.int32), (0, E_PAD - E))
    dst = jnp.pad(edge_index[1].astype(jnp.int32), (0, E_PAD - E))
    w = jnp.pad(edge_weight, (0, E_PAD - E))

    src2 = src.reshape(NROW2, CH)
    dst2 = dst.reshape(NROW2, CH)
    deg_f, deg_b = _deg_call(dst2, src2, w.reshape(NROW2, CH))
    wn_f2, wn_b2 = _wn_call(dst2, src2, w.reshape(NROW2, CH), deg_f, deg_b)
    h1 = _prop_call(h0, src2, dst2, wn_f2)
    h2 = _prop_call(h1, src2, dst2, wn_f2)
    h3 = _prop_call(h2, src2, dst2, wn_f2)
    h4 = _prop_call(h0, dst2, src2, wn_b2)
    h5 = _prop_call(h4, dst2, src2, wn_b2)
    h6 = _prop_call(h5, dst2, src2, wn_b2)

    wts = _prep_weights(conv_w, conv_b, lin_w, lin_b, mlp_w, mlp_b, ro_w, ro_b)
    embp = jnp.pad(emb, ((0, N_PAD - N), (0, 0)))
    out48 = _dense_call((h0, h1, h2, h3, h4, h5, h6), embp, *wts)

    out = out48[:N].reshape(N, BATCH, HORIZON).transpose(1, 2, 0)[..., None]
    return out
